# baseline (speedup 1.0000x reference)
import jax
import jax.numpy as jnp
from jax import lax
from jax.experimental import pallas as pl
from jax.experimental.pallas import tpu as pltpu
from jax.experimental.pallas import tpu_sc as plsc

B, S, D = 4, 8192, 128
FLAT = B * S
NC, NS = 2, 16
NW = NC * NS
PER_W = FLAT // NW
CHUNK = 64
N_CHUNKS = PER_W // CHUNK


NBUF = 10
GDEPTH = 8
SPAN = FLAT // NW // B
JCH = SPAN // CHUNK


def _emb_body(ids_hbm, tok_hbm, pos_hbm, out_hbm, idx_v, pos_sh, row_v,
              isem, psem, lsem, *gssems):
    gsems = list(gssems[:NBUF])
    ssems = list(gssems[NBUF:])
    sid = lax.axis_index("s")
    wid = sid * NC + lax.axis_index("c")
    s_base = wid * SPAN

    pos_stage = pltpu.async_copy(pos_hbm.at[pl.ds(s_base, SPAN)],
                                 pos_sh.at[sid], psem)
    idx_loads = [
        pltpu.async_copy(ids_hbm.at[b, pl.ds(s_base, SPAN)],
                         idx_v.at[pl.ds(b * SPAN, SPAN)], isem)
        for b in range(B)
    ]

    def chunk_coords(c):
        b, j = divmod(c, JCH)
        flat_base = b * S + s_base + j * CHUNK
        return b, j, flat_base

    def start_init(c):
        slot = c % NBUF
        _, j, _ = chunk_coords(c)
        return pltpu.async_copy(pos_sh.at[sid, pl.ds(j * CHUNK, CHUNK)],
                                row_v.at[slot], lsem)

    def start_store(c):
        slot = c % NBUF
        _, _, flat_base = chunk_coords(c)
        return pltpu.async_copy(row_v.at[slot],
                                out_hbm.at[pl.ds(flat_base, CHUNK)], ssems[slot])

    pos_stage.wait()
    for d in idx_loads:
        d.wait()
    inits = {0: start_init(0), 1: start_init(1)}
    gathers = {}
    stores = {}
    for c in range(N_CHUNKS):
        slot = c % NBUF
        b, j, _ = chunk_coords(c)
        inits.pop(c).wait()
        gathers[c] = pltpu.async_copy(
            tok_hbm.at[idx_v.at[pl.ds((b * JCH + j) * CHUNK, CHUNK)]],
            row_v.at[slot], gsems[slot], add=True)
        if c + 2 < N_CHUNKS:
            if c - GDEPTH >= 0:
                stores.pop(c - GDEPTH).wait()
            inits[c + 2] = start_init(c + 2)
        if c - (GDEPTH - 1) >= 0:
            p = c - (GDEPTH - 1)
            gathers.pop(p).wait()
            stores[p] = start_store(p)
    for p in range(N_CHUNKS - (GDEPTH - 1), N_CHUNKS):
        gathers.pop(p).wait()
        stores[p] = start_store(p)
    for c in sorted(stores):
        stores.pop(c).wait()


def kernel(token_ids, tok_emb, pos_emb):
    ids = token_ids.astype(jnp.int32)
    mesh = plsc.VectorSubcoreMesh(
        core_axis_name="c", subcore_axis_name="s",
        num_cores=NC, num_subcores=NS,
    )
    out = pl.kernel(
        _emb_body,
        out_type=jax.ShapeDtypeStruct((FLAT, D), jnp.float32),
        mesh=mesh,
        scratch_types=(
            [pltpu.VMEM((PER_W,), jnp.int32),
             pltpu.VMEM_SHARED((NS, SPAN, D), jnp.float32),
             pltpu.VMEM((NBUF, CHUNK, D), jnp.float32)]
            + [pltpu.SemaphoreType.DMA] * (3 + 2 * NBUF)
        ),
    )(ids, tok_emb, pos_emb)
    return out.reshape(B, S, D)


# speedup vs baseline: 1.0194x; 1.0194x over previous
import jax
import jax.numpy as jnp
from jax import lax
from jax.experimental import pallas as pl
from jax.experimental.pallas import tpu as pltpu
from jax.experimental.pallas import tpu_sc as plsc

B, S, D = 4, 8192, 128
FLAT = B * S
NC, NS = 2, 16
NW = NC * NS
PER_W = FLAT // NW
CHUNK = 128
N_CHUNKS = PER_W // CHUNK


NBUF = 5
GDEPTH = 3
SPAN = FLAT // NW // B
JCH = SPAN // CHUNK


def _emb_body(ids_hbm, tok_hbm, pos_hbm, out_hbm, idx_v, pos_sh, row_v,
              isem, psem, lsem, *gssems):
    gsems = list(gssems[:NBUF])
    ssems = list(gssems[NBUF:])
    sid = lax.axis_index("s")
    wid = sid * NC + lax.axis_index("c")
    s_base = wid * SPAN

    pos_stage = pltpu.async_copy(pos_hbm.at[pl.ds(s_base, SPAN)],
                                 pos_sh.at[sid], psem)
    idx_loads = [
        pltpu.async_copy(ids_hbm.at[b, pl.ds(s_base, SPAN)],
                         idx_v.at[pl.ds(b * SPAN, SPAN)], isem)
        for b in range(B)
    ]

    def chunk_coords(c):
        b, j = divmod(c, JCH)
        return b, j, s_base + j * CHUNK

    def start_init(c):
        slot = c % NBUF
        _, j, _ = chunk_coords(c)
        return pltpu.async_copy(pos_sh.at[sid, pl.ds(j * CHUNK, CHUNK)],
                                row_v.at[slot], lsem)

    def start_store(c):
        slot = c % NBUF
        b, _, srow = chunk_coords(c)
        return pltpu.async_copy(row_v.at[slot],
                                out_hbm.at[b, pl.ds(srow, CHUNK)], ssems[slot])

    pos_stage.wait()
    for d in idx_loads:
        d.wait()
    inits = {0: start_init(0), 1: start_init(1)}
    gathers = {}
    stores = {}
    for c in range(N_CHUNKS):
        slot = c % NBUF
        b, j, _ = chunk_coords(c)
        inits.pop(c).wait()
        gathers[c] = pltpu.async_copy(
            tok_hbm.at[idx_v.at[pl.ds((b * JCH + j) * CHUNK, CHUNK)]],
            row_v.at[slot], gsems[slot], add=True)
        if c + 2 < N_CHUNKS:
            if c - GDEPTH >= 0:
                stores.pop(c - GDEPTH).wait()
            inits[c + 2] = start_init(c + 2)
        if c - (GDEPTH - 1) >= 0:
            p = c - (GDEPTH - 1)
            gathers.pop(p).wait()
            stores[p] = start_store(p)
    for p in range(N_CHUNKS - (GDEPTH - 1), N_CHUNKS):
        gathers.pop(p).wait()
        stores[p] = start_store(p)
    for c in sorted(stores):
        stores.pop(c).wait()


def kernel(token_ids, tok_emb, pos_emb):
    ids = token_ids.astype(jnp.int32)
    mesh = plsc.VectorSubcoreMesh(
        core_axis_name="c", subcore_axis_name="s",
        num_cores=NC, num_subcores=NS,
    )
    out = pl.kernel(
        _emb_body,
        out_type=jax.ShapeDtypeStruct((B, S, D), jnp.float32),
        mesh=mesh,
        scratch_types=(
            [pltpu.VMEM((PER_W,), jnp.int32),
             pltpu.VMEM_SHARED((NS, SPAN, D), jnp.float32),
             pltpu.VMEM((NBUF, CHUNK, D), jnp.float32)]
            + [pltpu.SemaphoreType.DMA] * (3 + 2 * NBUF)
        ),
    )(ids, tok_emb, pos_emb)
    return out
